# baseline (speedup 1.0000x reference)
import functools

import jax
import jax.numpy as jnp
import numpy as np
from jax.experimental import pallas as pl
from jax.experimental.pallas import tpu as pltpu

B = 8
HRES = 32
WRES = 32
DIM = 768
HEADS = 12
WS = 8
MLP_HID = 3072
TOPK = 48
N = WS * WS
HD = DIM // HEADS
NWIN = (HRES // WS) * (WRES // WS) * B
ROWS = B * HRES * WRES
CHUNK = 512
WPC = CHUNK // N
GRID = ROWS // CHUNK


def _rel_index():
    coords = np.stack(np.meshgrid(np.arange(WS), np.arange(WS), indexing='ij'))
    cf = coords.reshape(2, -1)
    rel = (cf[:, :, None] - cf[:, None, :]).transpose(1, 2, 0).astype(np.int64)
    rel[:, :, 0] += WS - 1
    rel[:, :, 1] += WS - 1
    rel[:, :, 0] *= 2 * WS - 1
    return rel.sum(-1)


_REL_IDX = _rel_index()


def _ln_rows(x, g, b):
    mu = jnp.mean(x, axis=-1, keepdims=True)
    var = jnp.mean((x - mu) * (x - mu), axis=-1, keepdims=True)
    return (x - mu) * jax.lax.rsqrt(var + 1e-5) * g + b


def _erf(x):
    return jax.lax.erf(x)


def _gelu(x):
    return 0.5 * x * (1.0 + _erf(x * np.float32(1.0 / np.sqrt(2.0))))


def _block(x_ref, g1_ref, b1_ref, qkvw_ref, qkvb_ref, projw_ref, projb_ref,
           bias_ref, g2_ref, b2_ref, fc1w_ref, fc1b_ref, fc2w_ref, fc2b_ref,
           out_ref, y_ref):
    step = pl.program_id(0)
    xc = x_ref[...].reshape(2, WS, 4, WS, DIM) \
        .transpose(0, 2, 1, 3, 4).reshape(CHUNK, DIM)
    ln1 = _ln_rows(xc, g1_ref[...], b1_ref[...])
    ln1b = ln1.astype(jnp.bfloat16)
    qkv = jax.lax.dot(ln1b, qkvw_ref[...],
                      preferred_element_type=jnp.float32) + qkvb_ref[...]
    bias_t = bias_ref[...]

    scale = np.float32(HD ** -0.5)
    win_outs = []
    for w in range(WPC):
        base = w * N
        qb = qkv[base:base + N, 0:DIM].astype(jnp.bfloat16)
        kb = qkv[base:base + N, DIM:2 * DIM].astype(jnp.bfloat16)
        vb = qkv[base:base + N, 2 * DIM:3 * DIM].astype(jnp.bfloat16)
        heads = []
        for h in range(HEADS):
            sl = slice(h * HD, (h + 1) * HD)
            heads.append(jax.lax.dot_general(
                kb[:, sl], qb[:, sl], (((1,), (1,)), ((), ())),
                preferred_element_type=jnp.float32))
        attn = jnp.concatenate(heads, axis=1) * scale + bias_t
        cols = []
        for c in range(HEADS * N // 128):
            csl = slice(c * 128, (c + 1) * 128)
            ac = attn[:, csl]
            a = ac
            for _ in range(N - TOPK):
                cmin = jnp.min(a, axis=0, keepdims=True)
                a = jnp.where(a == cmin, jnp.inf, a)
            cmax = jnp.max(ac, axis=0, keepdims=True)
            p = jnp.where(a == jnp.inf, 0.0, jnp.exp(ac - cmax))
            s = jnp.maximum(jnp.sum(p, axis=0, keepdims=True), 1e-30)
            cols.append((p * (1.0 / s)).astype(jnp.bfloat16))
        wgt = jnp.concatenate(cols, axis=1)
        outs = []
        for h in range(HEADS):
            sl = slice(h * HD, (h + 1) * HD)
            outs.append(jax.lax.dot_general(
                wgt[:, h * N:(h + 1) * N], vb[:, sl], (((0,), (0,)), ((), ())),
                preferred_element_type=jnp.float32))
        win_outs.append(jnp.concatenate(outs, axis=1))
    att = jnp.concatenate(win_outs, axis=0).astype(jnp.bfloat16)
    y = jax.lax.dot(att, projw_ref[...],
                    preferred_element_type=jnp.float32) + projb_ref[...] + xc
    y_ref[step % 2] = y
    yp = y_ref[(step + 1) % 2]
    ln2b = _ln_rows(yp, g2_ref[...], b2_ref[...]).astype(jnp.bfloat16)
    h1 = jax.lax.dot(ln2b, fc1w_ref[...],
                     preferred_element_type=jnp.float32) + fc1b_ref[...]
    hb = _gelu(h1).astype(jnp.bfloat16)
    out = jax.lax.dot(hb, fc2w_ref[...],
                      preferred_element_type=jnp.float32) + fc2b_ref[...] + yp
    out_ref[...] = out.reshape(2, 4, WS, WS, DIM) \
        .transpose(0, 2, 1, 3, 4).reshape(CHUNK, DIM)


@jax.jit
def kernel(x, norm1_g, norm1_b, qkv_w, qkv_b, proj_w, proj_b, rel_bias,
           norm2_g, norm2_b, fc1_w, fc1_b, fc2_w, fc2_b):
    xw = x.reshape(ROWS, DIM)
    rbg = rel_bias[jnp.asarray(_REL_IDX.reshape(-1))].reshape(N, N, HEADS)
    bias_t = jnp.transpose(rbg, (1, 2, 0)).reshape(N, HEADS * N)

    full = lambda shape: pl.BlockSpec(shape, lambda i: (0,) * len(shape))
    row2 = lambda v: v.reshape(1, -1)

    out = pl.pallas_call(
        _block,
        grid=(GRID + 1,),
        in_specs=[
            pl.BlockSpec((CHUNK, DIM), lambda i: (jnp.minimum(i, GRID - 1), 0)),
            full((1, DIM)), full((1, DIM)),
            full((DIM, 3 * DIM)), full((1, 3 * DIM)),
            full((DIM, DIM)), full((1, DIM)),
            full((N, HEADS * N)),
            full((1, DIM)), full((1, DIM)),
            full((DIM, MLP_HID)), full((1, MLP_HID)),
            full((MLP_HID, DIM)), full((1, DIM)),
        ],
        out_specs=pl.BlockSpec((CHUNK, DIM), lambda i: (jnp.maximum(i - 1, 0), 0)),
        out_shape=jax.ShapeDtypeStruct((ROWS, DIM), jnp.float32),
        scratch_shapes=[pltpu.VMEM((2, CHUNK, DIM), jnp.float32)],
        compiler_params=pltpu.CompilerParams(
            dimension_semantics=("arbitrary",),
            vmem_limit_bytes=100 * 1024 * 1024,
        ),
    )(xw, row2(norm1_g), row2(norm1_b),
      qkv_w.astype(jnp.bfloat16), row2(qkv_b),
      proj_w.astype(jnp.bfloat16), row2(proj_b),
      bias_t,
      row2(norm2_g), row2(norm2_b),
      fc1_w.astype(jnp.bfloat16), row2(fc1_b),
      fc2_w.astype(jnp.bfloat16), row2(fc2_b))

    return out.reshape(B, HRES * WRES, DIM)


# speedup vs baseline: 1.1524x; 1.1524x over previous
import functools

import jax
import jax.numpy as jnp
import numpy as np
from jax.experimental import pallas as pl
from jax.experimental.pallas import tpu as pltpu

B = 8
HRES = 32
WRES = 32
DIM = 768
HEADS = 12
WS = 8
MLP_HID = 3072
TOPK = 48
N = WS * WS
HD = DIM // HEADS
NWIN = (HRES // WS) * (WRES // WS) * B
ROWS = B * HRES * WRES
CHUNK = 512
WPC = CHUNK // N
GRID = ROWS // CHUNK


def _rel_index():
    coords = np.stack(np.meshgrid(np.arange(WS), np.arange(WS), indexing='ij'))
    cf = coords.reshape(2, -1)
    rel = (cf[:, :, None] - cf[:, None, :]).transpose(1, 2, 0).astype(np.int64)
    rel[:, :, 0] += WS - 1
    rel[:, :, 1] += WS - 1
    rel[:, :, 0] *= 2 * WS - 1
    return rel.sum(-1)


_REL_IDX = _rel_index()


def _ln_rows(x, g, b):
    mu = jnp.mean(x, axis=-1, keepdims=True)
    var = jnp.mean((x - mu) * (x - mu), axis=-1, keepdims=True)
    return (x - mu) * jax.lax.rsqrt(var + 1e-5) * g + b


def _erf(x):
    return jax.lax.erf(x)


def _gelu(x):
    return 0.5 * x * (1.0 + _erf(x * np.float32(1.0 / np.sqrt(2.0))))


def _block(x_ref, g1_ref, b1_ref, qkvw_ref, qkvb_ref, projw_ref, projb_ref,
           bias_ref, g2_ref, b2_ref, fc1w_ref, fc1b_ref, fc2w_ref, fc2b_ref,
           out_ref):
    xc = x_ref[...].reshape(2, WS, 4, WS, DIM) \
        .transpose(0, 2, 1, 3, 4).reshape(CHUNK, DIM)
    ln1 = _ln_rows(xc, g1_ref[...], b1_ref[...])
    ln1b = ln1.astype(jnp.bfloat16)
    qkv = jax.lax.dot(ln1b, qkvw_ref[...],
                      preferred_element_type=jnp.float32) + qkvb_ref[...]
    bias_t = bias_ref[...]

    scale = np.float32(HD ** -0.5)

    def _dots(w):
        base = w * N
        qb = qkv[base:base + N, 0:DIM].astype(jnp.bfloat16)
        kb = qkv[base:base + N, DIM:2 * DIM].astype(jnp.bfloat16)
        heads = []
        for h in range(HEADS):
            sl = slice(h * HD, (h + 1) * HD)
            heads.append(jax.lax.dot_general(
                kb[:, sl], qb[:, sl], (((1,), (1,)), ((), ())),
                preferred_element_type=jnp.float32))
        return jnp.concatenate(heads, axis=1) * scale + bias_t

    def _select(attn):
        cols = []
        for c in range(HEADS * N // 128):
            ac = attn[:, c * 128:(c + 1) * 128]
            a = ac
            for _ in range(N - TOPK):
                cmin = jnp.min(a, axis=0, keepdims=True)
                a = jnp.where(a == cmin, jnp.inf, a)
            cmax = jnp.max(ac, axis=0, keepdims=True)
            p = jnp.where(a == jnp.inf, 0.0, jnp.exp(ac - cmax))
            s = jnp.maximum(jnp.sum(p, axis=0, keepdims=True), 1e-30)
            cols.append((p * (1.0 / s)).astype(jnp.bfloat16))
        return jnp.concatenate(cols, axis=1)

    def _wv(wgt, w):
        base = w * N
        vb = qkv[base:base + N, 2 * DIM:3 * DIM].astype(jnp.bfloat16)
        outs = []
        for h in range(HEADS):
            sl = slice(h * HD, (h + 1) * HD)
            outs.append(jax.lax.dot_general(
                wgt[:, h * N:(h + 1) * N], vb[:, sl], (((0,), (0,)), ((), ())),
                preferred_element_type=jnp.float32))
        return jnp.concatenate(outs, axis=1)

    win_outs = []
    attn_prev = _dots(0)
    for w in range(1, WPC + 1):
        attn_next = _dots(w) if w < WPC else None
        wgt = _select(attn_prev)
        win_outs.append(_wv(wgt, w - 1))
        attn_prev = attn_next
    att = jnp.concatenate(win_outs, axis=0).astype(jnp.bfloat16)
    y = jax.lax.dot(att, projw_ref[...],
                    preferred_element_type=jnp.float32) + projb_ref[...] + xc
    ln2b = _ln_rows(y, g2_ref[...], b2_ref[...]).astype(jnp.bfloat16)
    h1 = jax.lax.dot(ln2b, fc1w_ref[...],
                     preferred_element_type=jnp.float32) + fc1b_ref[...]
    hb = _gelu(h1).astype(jnp.bfloat16)
    out = jax.lax.dot(hb, fc2w_ref[...],
                      preferred_element_type=jnp.float32) + fc2b_ref[...] + y
    out_ref[...] = out.reshape(2, 4, WS, WS, DIM) \
        .transpose(0, 2, 1, 3, 4).reshape(CHUNK, DIM)


@jax.jit
def kernel(x, norm1_g, norm1_b, qkv_w, qkv_b, proj_w, proj_b, rel_bias,
           norm2_g, norm2_b, fc1_w, fc1_b, fc2_w, fc2_b):
    xw = x.reshape(ROWS, DIM)
    rbg = rel_bias[jnp.asarray(_REL_IDX.reshape(-1))].reshape(N, N, HEADS)
    bias_t = jnp.transpose(rbg, (1, 2, 0)).reshape(N, HEADS * N)

    full = lambda shape: pl.BlockSpec(shape, lambda i: (0,) * len(shape))
    row2 = lambda v: v.reshape(1, -1)

    out = pl.pallas_call(
        _block,
        grid=(GRID,),
        in_specs=[
            pl.BlockSpec((CHUNK, DIM), lambda i: (i, 0)),
            full((1, DIM)), full((1, DIM)),
            full((DIM, 3 * DIM)), full((1, 3 * DIM)),
            full((DIM, DIM)), full((1, DIM)),
            full((N, HEADS * N)),
            full((1, DIM)), full((1, DIM)),
            full((DIM, MLP_HID)), full((1, MLP_HID)),
            full((MLP_HID, DIM)), full((1, DIM)),
        ],
        out_specs=pl.BlockSpec((CHUNK, DIM), lambda i: (i, 0)),
        out_shape=jax.ShapeDtypeStruct((ROWS, DIM), jnp.float32),
        compiler_params=pltpu.CompilerParams(
            dimension_semantics=("arbitrary",),
            vmem_limit_bytes=100 * 1024 * 1024,
        ),
    )(xw, row2(norm1_g), row2(norm1_b),
      qkv_w.astype(jnp.bfloat16), row2(qkv_b),
      proj_w.astype(jnp.bfloat16), row2(proj_b),
      bias_t,
      row2(norm2_g), row2(norm2_b),
      fc1_w.astype(jnp.bfloat16), row2(fc1_b),
      fc2_w.astype(jnp.bfloat16), row2(fc2_b))

    return out.reshape(B, HRES * WRES, DIM)


# speedup vs baseline: 1.1691x; 1.0145x over previous
import functools

import jax
import jax.numpy as jnp
import numpy as np
from jax.experimental import pallas as pl
from jax.experimental.pallas import tpu as pltpu

B = 8
HRES = 32
WRES = 32
DIM = 768
HEADS = 12
WS = 8
MLP_HID = 3072
TOPK = 48
N = WS * WS
HD = DIM // HEADS
NWIN = (HRES // WS) * (WRES // WS) * B
ROWS = B * HRES * WRES
CHUNK = 1024
WPC = CHUNK // N
GRID = ROWS // CHUNK


def _rel_index():
    coords = np.stack(np.meshgrid(np.arange(WS), np.arange(WS), indexing='ij'))
    cf = coords.reshape(2, -1)
    rel = (cf[:, :, None] - cf[:, None, :]).transpose(1, 2, 0).astype(np.int64)
    rel[:, :, 0] += WS - 1
    rel[:, :, 1] += WS - 1
    rel[:, :, 0] *= 2 * WS - 1
    return rel.sum(-1)


_REL_IDX = _rel_index()


def _ln_rows(x, g, b):
    mu = jnp.mean(x, axis=-1, keepdims=True)
    var = jnp.mean((x - mu) * (x - mu), axis=-1, keepdims=True)
    return (x - mu) * jax.lax.rsqrt(var + 1e-5) * g + b


def _gelu(x):
    return 0.5 * x * (1.0 + jax.lax.erf(x * np.float32(1.0 / np.sqrt(2.0))))


def _block(x_ref, g1_ref, b1_ref, qkvw_ref, qkvb_ref, projw_ref, projb_ref,
           bias_ref, g2_ref, b2_ref, fc1w_ref, fc1b_ref, fc2w_ref, fc2b_ref,
           out_ref):
    ng = CHUNK // (WS * WRES)
    xc = x_ref[...].reshape(ng, WS, 4, WS, DIM) \
        .transpose(0, 2, 1, 3, 4).reshape(CHUNK, DIM)
    ln1 = _ln_rows(xc, g1_ref[...], b1_ref[...])
    ln1b = ln1.astype(jnp.bfloat16)
    qkv = jax.lax.dot(ln1b, qkvw_ref[...],
                      preferred_element_type=jnp.float32) + qkvb_ref[...]
    bias_t = bias_ref[...]

    scale = np.float32(HD ** -0.5)

    def _dots(w):
        base = w * N
        qb = qkv[base:base + N, 0:DIM].astype(jnp.bfloat16)
        kb = qkv[base:base + N, DIM:2 * DIM].astype(jnp.bfloat16)
        heads = []
        for h in range(HEADS):
            sl = slice(h * HD, (h + 1) * HD)
            heads.append(jax.lax.dot_general(
                kb[:, sl], qb[:, sl], (((1,), (1,)), ((), ())),
                preferred_element_type=jnp.float32))
        return jnp.concatenate(heads, axis=1) * scale + bias_t

    def _select(attn):
        cols = []
        for c in range(HEADS * N // 128):
            ac = attn[:, c * 128:(c + 1) * 128]
            a = ac
            for _ in range(N - TOPK):
                cmin = jnp.min(a, axis=0, keepdims=True)
                a = jnp.where(a == cmin, jnp.inf, a)
            cmax = jnp.max(ac, axis=0, keepdims=True)
            p = jnp.where(a == jnp.inf, 0.0, jnp.exp(ac - cmax))
            s = jnp.maximum(jnp.sum(p, axis=0, keepdims=True), 1e-30)
            cols.append((p * (1.0 / s)).astype(jnp.bfloat16))
        return jnp.concatenate(cols, axis=1)

    def _wv(wgt, w):
        base = w * N
        vb = qkv[base:base + N, 2 * DIM:3 * DIM].astype(jnp.bfloat16)
        outs = []
        for h in range(HEADS):
            sl = slice(h * HD, (h + 1) * HD)
            outs.append(jax.lax.dot_general(
                wgt[:, h * N:(h + 1) * N], vb[:, sl], (((0,), (0,)), ((), ())),
                preferred_element_type=jnp.float32))
        return jnp.concatenate(outs, axis=1)

    win_outs = []
    attn_prev = _dots(0)
    for w in range(1, WPC + 1):
        attn_next = _dots(w) if w < WPC else None
        wgt = _select(attn_prev)
        win_outs.append(_wv(wgt, w - 1))
        attn_prev = attn_next
    att = jnp.concatenate(win_outs, axis=0).astype(jnp.bfloat16)
    y = jax.lax.dot(att, projw_ref[...],
                    preferred_element_type=jnp.float32) + projb_ref[...] + xc
    ln2b = _ln_rows(y, g2_ref[...], b2_ref[...]).astype(jnp.bfloat16)
    h1 = jax.lax.dot(ln2b, fc1w_ref[...],
                     preferred_element_type=jnp.float32) + fc1b_ref[...]
    hb = _gelu(h1).astype(jnp.bfloat16)
    out = jax.lax.dot(hb, fc2w_ref[...],
                      preferred_element_type=jnp.float32) + fc2b_ref[...] + y
    out_ref[...] = out.reshape(ng, 4, WS, WS, DIM) \
        .transpose(0, 2, 1, 3, 4).reshape(CHUNK, DIM)


@jax.jit
def kernel(x, norm1_g, norm1_b, qkv_w, qkv_b, proj_w, proj_b, rel_bias,
           norm2_g, norm2_b, fc1_w, fc1_b, fc2_w, fc2_b):
    xw = x.reshape(ROWS, DIM)
    rbg = rel_bias[jnp.asarray(_REL_IDX.reshape(-1))].reshape(N, N, HEADS)
    bias_t = jnp.transpose(rbg, (1, 2, 0)).reshape(N, HEADS * N)

    full = lambda shape: pl.BlockSpec(shape, lambda i: (0,) * len(shape))
    row2 = lambda v: v.reshape(1, -1)

    out = pl.pallas_call(
        _block,
        grid=(GRID,),
        in_specs=[
            pl.BlockSpec((CHUNK, DIM), lambda i: (i, 0)),
            full((1, DIM)), full((1, DIM)),
            full((DIM, 3 * DIM)), full((1, 3 * DIM)),
            full((DIM, DIM)), full((1, DIM)),
            full((N, HEADS * N)),
            full((1, DIM)), full((1, DIM)),
            full((DIM, MLP_HID)), full((1, MLP_HID)),
            full((MLP_HID, DIM)), full((1, DIM)),
        ],
        out_specs=pl.BlockSpec((CHUNK, DIM), lambda i: (i, 0)),
        out_shape=jax.ShapeDtypeStruct((ROWS, DIM), jnp.float32),
        compiler_params=pltpu.CompilerParams(
            dimension_semantics=("arbitrary",),
            vmem_limit_bytes=100 * 1024 * 1024,
        ),
    )(xw, row2(norm1_g), row2(norm1_b),
      qkv_w.astype(jnp.bfloat16), row2(qkv_b),
      proj_w.astype(jnp.bfloat16), row2(proj_b),
      bias_t,
      row2(norm2_g), row2(norm2_b),
      fc1_w.astype(jnp.bfloat16), row2(fc1_b),
      fc2_w.astype(jnp.bfloat16), row2(fc2_b))

    return out.reshape(B, HRES * WRES, DIM)


# speedup vs baseline: 1.1922x; 1.0197x over previous
import functools

import jax
import jax.numpy as jnp
import numpy as np
from jax.experimental import pallas as pl
from jax.experimental.pallas import tpu as pltpu

B = 8
HRES = 32
WRES = 32
DIM = 768
HEADS = 12
WS = 8
MLP_HID = 3072
TOPK = 48
N = WS * WS
HD = DIM // HEADS
NWIN = (HRES // WS) * (WRES // WS) * B
ROWS = B * HRES * WRES
CHUNK = 1024
WPC = CHUNK // N
GRID = ROWS // CHUNK


def _rel_index():
    coords = np.stack(np.meshgrid(np.arange(WS), np.arange(WS), indexing='ij'))
    cf = coords.reshape(2, -1)
    rel = (cf[:, :, None] - cf[:, None, :]).transpose(1, 2, 0).astype(np.int64)
    rel[:, :, 0] += WS - 1
    rel[:, :, 1] += WS - 1
    rel[:, :, 0] *= 2 * WS - 1
    return rel.sum(-1)


_REL_IDX = _rel_index()


def _ln_rows(x, g, b):
    mu = jnp.mean(x, axis=-1, keepdims=True)
    var = jnp.mean((x - mu) * (x - mu), axis=-1, keepdims=True)
    return (x - mu) * jax.lax.rsqrt(var + 1e-5) * g + b


def _gelu(x):
    return 0.5 * x * (1.0 + jax.lax.erf(x * np.float32(1.0 / np.sqrt(2.0))))


def _block(x_ref, g1_ref, b1_ref, qkvw_ref, qkvb_ref, projw_ref, projb_ref,
           bias_ref, g2_ref, b2_ref, fc1w_ref, fc1b_ref, fc2w_ref, fc2b_ref,
           out_ref):
    ng = CHUNK // (WS * WRES)
    xc = x_ref[...].reshape(ng, WS, 4, WS, DIM) \
        .transpose(0, 2, 1, 3, 4).reshape(CHUNK, DIM)
    ln1 = _ln_rows(xc, g1_ref[...], b1_ref[...])
    ln1b = ln1.astype(jnp.bfloat16)
    bias_t = bias_ref[...]
    scale = np.float32(HD ** -0.5)
    HW = WPC // 2
    HR = CHUNK // 2

    def _dots(qkv, w):
        base = w * N
        qb = qkv[base:base + N, 0:DIM].astype(jnp.bfloat16)
        kb = qkv[base:base + N, DIM:2 * DIM].astype(jnp.bfloat16)
        heads = []
        for h in range(HEADS):
            sl = slice(h * HD, (h + 1) * HD)
            heads.append(jax.lax.dot_general(
                kb[:, sl], qb[:, sl], (((1,), (1,)), ((), ())),
                preferred_element_type=jnp.float32))
        return jnp.concatenate(heads, axis=1) * scale + bias_t

    def _select(attn):
        cols = []
        for c in range(HEADS * N // 128):
            ac = attn[:, c * 128:(c + 1) * 128]
            a = ac
            for _ in range(N - TOPK):
                cmin = jnp.min(a, axis=0, keepdims=True)
                a = jnp.where(a == cmin, jnp.inf, a)
            cmax = jnp.max(ac, axis=0, keepdims=True)
            p = jnp.where(a == jnp.inf, 0.0, jnp.exp(ac - cmax))
            s = jnp.maximum(jnp.sum(p, axis=0, keepdims=True), 1e-30)
            cols.append((p * (1.0 / s)).astype(jnp.bfloat16))
        return jnp.concatenate(cols, axis=1)

    def _wv(qkv, wgt, w):
        base = w * N
        vb = qkv[base:base + N, 2 * DIM:3 * DIM].astype(jnp.bfloat16)
        outs = []
        for h in range(HEADS):
            sl = slice(h * HD, (h + 1) * HD)
            outs.append(jax.lax.dot_general(
                wgt[:, h * N:(h + 1) * N], vb[:, sl], (((0,), (0,)), ((), ())),
                preferred_element_type=jnp.float32))
        return jnp.concatenate(outs, axis=1)

    def _attn_half(qkv, extra):
        win_outs = []
        attn_prev = _dots(qkv, 0)
        for w in range(1, HW + 1):
            attn_next = _dots(qkv, w) if w < HW else None
            if extra is not None:
                extra(w - 1)
            wgt = _select(attn_prev)
            win_outs.append(_wv(qkv, wgt, w - 1))
            attn_prev = attn_next
        return jnp.concatenate(win_outs, axis=0).astype(jnp.bfloat16)

    qkv_a = jax.lax.dot(ln1b[:HR], qkvw_ref[...],
                        preferred_element_type=jnp.float32) + qkvb_ref[...]
    qkv_b = jax.lax.dot(ln1b[HR:], qkvw_ref[...],
                        preferred_element_type=jnp.float32) + qkvb_ref[...]
    att_a = _attn_half(qkv_a, None)
    y_a = jax.lax.dot(att_a, projw_ref[...],
                      preferred_element_type=jnp.float32) \
        + projb_ref[...] + xc[:HR]
    ln2b_a = _ln_rows(y_a, g2_ref[...], b2_ref[...]).astype(jnp.bfloat16)

    STRIP = MLP_HID // HW
    gel_a = []

    def _mlp_strip(w):
        st = w * STRIP
        h1s = jax.lax.dot(ln2b_a, fc1w_ref[:, st:st + STRIP],
                          preferred_element_type=jnp.float32) \
            + fc1b_ref[:, st:st + STRIP]
        gel_a.append(_gelu(h1s).astype(jnp.bfloat16))

    att_b = _attn_half(qkv_b, _mlp_strip)
    y_b = jax.lax.dot(att_b, projw_ref[...],
                      preferred_element_type=jnp.float32) \
        + projb_ref[...] + xc[HR:]
    hb_a = jnp.concatenate(gel_a, axis=1)
    out_a = jax.lax.dot(hb_a, fc2w_ref[...],
                        preferred_element_type=jnp.float32) \
        + fc2b_ref[...] + y_a
    ln2b_b = _ln_rows(y_b, g2_ref[...], b2_ref[...]).astype(jnp.bfloat16)
    h1_b = jax.lax.dot(ln2b_b, fc1w_ref[...],
                       preferred_element_type=jnp.float32) + fc1b_ref[...]
    hb_b = _gelu(h1_b).astype(jnp.bfloat16)
    out_b = jax.lax.dot(hb_b, fc2w_ref[...],
                        preferred_element_type=jnp.float32) \
        + fc2b_ref[...] + y_b
    out = jnp.concatenate([out_a, out_b], axis=0)
    out_ref[...] = out.reshape(ng, 4, WS, WS, DIM) \
        .transpose(0, 2, 1, 3, 4).reshape(CHUNK, DIM)


@jax.jit
def kernel(x, norm1_g, norm1_b, qkv_w, qkv_b, proj_w, proj_b, rel_bias,
           norm2_g, norm2_b, fc1_w, fc1_b, fc2_w, fc2_b):
    xw = x.reshape(ROWS, DIM)
    rbg = rel_bias[jnp.asarray(_REL_IDX.reshape(-1))].reshape(N, N, HEADS)
    bias_t = jnp.transpose(rbg, (1, 2, 0)).reshape(N, HEADS * N)

    full = lambda shape: pl.BlockSpec(shape, lambda i: (0,) * len(shape))
    row2 = lambda v: v.reshape(1, -1)

    out = pl.pallas_call(
        _block,
        grid=(GRID,),
        in_specs=[
            pl.BlockSpec((CHUNK, DIM), lambda i: (i, 0)),
            full((1, DIM)), full((1, DIM)),
            full((DIM, 3 * DIM)), full((1, 3 * DIM)),
            full((DIM, DIM)), full((1, DIM)),
            full((N, HEADS * N)),
            full((1, DIM)), full((1, DIM)),
            full((DIM, MLP_HID)), full((1, MLP_HID)),
            full((MLP_HID, DIM)), full((1, DIM)),
        ],
        out_specs=pl.BlockSpec((CHUNK, DIM), lambda i: (i, 0)),
        out_shape=jax.ShapeDtypeStruct((ROWS, DIM), jnp.float32),
        compiler_params=pltpu.CompilerParams(
            dimension_semantics=("arbitrary",),
            vmem_limit_bytes=100 * 1024 * 1024,
        ),
    )(xw, row2(norm1_g), row2(norm1_b),
      qkv_w.astype(jnp.bfloat16), row2(qkv_b),
      proj_w.astype(jnp.bfloat16), row2(proj_b),
      bias_t,
      row2(norm2_g), row2(norm2_b),
      fc1_w.astype(jnp.bfloat16), row2(fc1_b),
      fc2_w.astype(jnp.bfloat16), row2(fc2_b))

    return out.reshape(B, HRES * WRES, DIM)


# speedup vs baseline: 1.1962x; 1.0034x over previous
import functools

import jax
import jax.numpy as jnp
import numpy as np
from jax.experimental import pallas as pl
from jax.experimental.pallas import tpu as pltpu

B = 8
HRES = 32
WRES = 32
DIM = 768
HEADS = 12
WS = 8
MLP_HID = 3072
TOPK = 48
N = WS * WS
HD = DIM // HEADS
NWIN = (HRES // WS) * (WRES // WS) * B
ROWS = B * HRES * WRES
CHUNK = 512
WPC = CHUNK // N
GRID = ROWS // CHUNK


def _rel_index():
    coords = np.stack(np.meshgrid(np.arange(WS), np.arange(WS), indexing='ij'))
    cf = coords.reshape(2, -1)
    rel = (cf[:, :, None] - cf[:, None, :]).transpose(1, 2, 0).astype(np.int64)
    rel[:, :, 0] += WS - 1
    rel[:, :, 1] += WS - 1
    rel[:, :, 0] *= 2 * WS - 1
    return rel.sum(-1)


_REL_IDX = _rel_index()


def _ln_rows(x, g, b):
    mu = jnp.mean(x, axis=-1, keepdims=True)
    var = jnp.mean((x - mu) * (x - mu), axis=-1, keepdims=True)
    return (x - mu) * jax.lax.rsqrt(var + 1e-5) * g + b


def _gelu(x):
    return 0.5 * x * (1.0 + jax.lax.erf(x * np.float32(1.0 / np.sqrt(2.0))))


def _block(x_ref, g1_ref, b1_ref, qkvw_ref, qkvb_ref, projw_ref, projb_ref,
           bias_ref, g2_ref, b2_ref, fc1w_ref, fc1b_ref, fc2w_ref, fc2b_ref,
           out_ref, y_ref):
    ng = CHUNK // (WS * WRES)
    xc = x_ref[...].reshape(ng, WS, 4, WS, DIM) \
        .transpose(0, 2, 1, 3, 4).reshape(CHUNK, DIM)
    ln1 = _ln_rows(xc, g1_ref[...], b1_ref[...])
    ln1b = ln1.astype(jnp.bfloat16)
    bias_t = bias_ref[...]
    scale = np.float32(HD ** -0.5)

    def _dots(qkv, w):
        base = w * N
        qb = qkv[base:base + N, 0:DIM].astype(jnp.bfloat16)
        kb = qkv[base:base + N, DIM:2 * DIM].astype(jnp.bfloat16)
        heads = []
        for h in range(HEADS):
            sl = slice(h * HD, (h + 1) * HD)
            heads.append(jax.lax.dot_general(
                kb[:, sl], qb[:, sl], (((1,), (1,)), ((), ())),
                preferred_element_type=jnp.float32))
        return jnp.concatenate(heads, axis=1) * scale + bias_t

    def _select(attn):
        cols = []
        for c in range(HEADS * N // 128):
            ac = attn[:, c * 128:(c + 1) * 128]
            a = ac
            for _ in range(N - TOPK):
                cmin = jnp.min(a, axis=0, keepdims=True)
                a = jnp.where(a == cmin, jnp.inf, a)
            cmax = jnp.max(ac, axis=0, keepdims=True)
            p = jnp.where(a == jnp.inf, 0.0, jnp.exp(ac - cmax))
            s = jnp.maximum(jnp.sum(p, axis=0, keepdims=True), 1e-30)
            cols.append((p * (1.0 / s)).astype(jnp.bfloat16))
        return jnp.concatenate(cols, axis=1)

    def _wv(qkv, wgt, w):
        base = w * N
        vb = qkv[base:base + N, 2 * DIM:3 * DIM].astype(jnp.bfloat16)
        outs = []
        for h in range(HEADS):
            sl = slice(h * HD, (h + 1) * HD)
            outs.append(jax.lax.dot_general(
                wgt[:, h * N:(h + 1) * N], vb[:, sl], (((0,), (0,)), ((), ())),
                preferred_element_type=jnp.float32))
        return jnp.concatenate(outs, axis=1)

    qkv = jax.lax.dot(ln1b, qkvw_ref[...],
                      preferred_element_type=jnp.float32) + qkvb_ref[...]
    yp = y_ref[...]
    ln2b_p = _ln_rows(yp, g2_ref[...], b2_ref[...]).astype(jnp.bfloat16)
    STRIP = MLP_HID // WPC
    gels = []

    def _mlp_strip(w):
        st = w * STRIP
        h1s = jax.lax.dot(ln2b_p, fc1w_ref[:, st:st + STRIP],
                          preferred_element_type=jnp.float32) \
            + fc1b_ref[:, st:st + STRIP]
        gels.append(_gelu(h1s).astype(jnp.bfloat16))

    win_outs = []
    attn_prev = _dots(qkv, 0)
    for w in range(1, WPC + 1):
        attn_next = _dots(qkv, w) if w < WPC else None
        _mlp_strip(w - 1)
        wgt = _select(attn_prev)
        win_outs.append(_wv(qkv, wgt, w - 1))
        attn_prev = attn_next
    att = jnp.concatenate(win_outs, axis=0).astype(jnp.bfloat16)

    hbp = jnp.concatenate(gels, axis=1)
    outp = jax.lax.dot(hbp, fc2w_ref[...],
                       preferred_element_type=jnp.float32) \
        + fc2b_ref[...] + yp
    out_ref[...] = outp.reshape(ng, 4, WS, WS, DIM) \
        .transpose(0, 2, 1, 3, 4).reshape(CHUNK, DIM)
    y = jax.lax.dot(att, projw_ref[...],
                    preferred_element_type=jnp.float32) + projb_ref[...] + xc
    y_ref[...] = y


@jax.jit
def kernel(x, norm1_g, norm1_b, qkv_w, qkv_b, proj_w, proj_b, rel_bias,
           norm2_g, norm2_b, fc1_w, fc1_b, fc2_w, fc2_b):
    xw = x.reshape(ROWS, DIM)
    rbg = rel_bias[jnp.asarray(_REL_IDX.reshape(-1))].reshape(N, N, HEADS)
    bias_t = jnp.transpose(rbg, (1, 2, 0)).reshape(N, HEADS * N)

    full = lambda shape: pl.BlockSpec(shape, lambda i: (0,) * len(shape))
    row2 = lambda v: v.reshape(1, -1)

    out = pl.pallas_call(
        _block,
        grid=(GRID + 1,),
        in_specs=[
            pl.BlockSpec((CHUNK, DIM), lambda i: (jnp.minimum(i, GRID - 1), 0)),
            full((1, DIM)), full((1, DIM)),
            full((DIM, 3 * DIM)), full((1, 3 * DIM)),
            full((DIM, DIM)), full((1, DIM)),
            full((N, HEADS * N)),
            full((1, DIM)), full((1, DIM)),
            full((DIM, MLP_HID)), full((1, MLP_HID)),
            full((MLP_HID, DIM)), full((1, DIM)),
        ],
        out_specs=pl.BlockSpec((CHUNK, DIM), lambda i: (jnp.maximum(i - 1, 0), 0)),
        out_shape=jax.ShapeDtypeStruct((ROWS, DIM), jnp.float32),
        scratch_shapes=[pltpu.VMEM((CHUNK, DIM), jnp.float32)],
        compiler_params=pltpu.CompilerParams(
            dimension_semantics=("arbitrary",),
            vmem_limit_bytes=100 * 1024 * 1024,
        ),
    )(xw, row2(norm1_g), row2(norm1_b),
      qkv_w.astype(jnp.bfloat16), row2(qkv_b),
      proj_w.astype(jnp.bfloat16), row2(proj_b),
      bias_t,
      row2(norm2_g), row2(norm2_b),
      fc1_w.astype(jnp.bfloat16), row2(fc1_b),
      fc2_w.astype(jnp.bfloat16), row2(fc2_b))

    return out.reshape(B, HRES * WRES, DIM)
